# baseline (speedup 1.0000x reference)
import jax
import jax.numpy as jnp
from jax import lax
from jax.experimental import pallas as pl
from jax.experimental.pallas import tpu as pltpu, tpu_sc as plsc

NUM_CORES = 2
NUM_SUBCORES = 16
NW = NUM_CORES * NUM_SUBCORES
BATCH = 16384
EMBED = 32
B_PER_W = BATCH // NW
CHUNK = 128
NCHUNK = B_PER_W // CHUNK
GROUPS = B_PER_W // 16

_mesh = plsc.VectorSubcoreMesh(core_axis_name="c", subcore_axis_name="s",
                               num_cores=NUM_CORES, num_subcores=NUM_SUBCORES)


def _sc_body(users_r, items_r, ue_r, ie_r, out_r,
             uidx_v, iidx_v, urows_v, irows_v, out_v, sem):
    wid = lax.axis_index("s") * NUM_CORES + lax.axis_index("c")

    pltpu.sync_copy(users_r.at[wid], uidx_v)
    pltpu.sync_copy(items_r.at[wid], iidx_v)

    copies = []
    for ch in range(NCHUNK):
        copies.append(pltpu.async_copy(ue_r.at[uidx_v.at[ch]], urows_v.at[ch], sem))
        copies.append(pltpu.async_copy(ie_r.at[iidx_v.at[ch]], irows_v.at[ch], sem))
    for cp in copies:
        cp.wait()

    lanes = lax.iota(jnp.int32, 16)

    def group(j, _):
        b = j * 16 + lanes
        ch = lax.shift_right_logical(b, 7)
        r = lax.bitwise_and(b, 127)
        acc = jnp.zeros((16,), jnp.float32)
        for d in range(EMBED):
            dv = jnp.full((16,), d, jnp.int32)
            u = plsc.load_gather(urows_v, [ch, r, dv])
            v = plsc.load_gather(irows_v, [ch, r, dv])
            acc = acc + u * v
        out_v[pl.ds(pl.multiple_of(j * 16, 16), 16)] = acc
        return 0

    lax.fori_loop(0, GROUPS, group, 0)
    pltpu.sync_copy(out_v, out_r.at[wid])


_sc_kernel = pl.kernel(
    _sc_body,
    out_type=jax.ShapeDtypeStruct((NW, B_PER_W), jnp.float32),
    mesh=_mesh,
    compiler_params=pltpu.CompilerParams(needs_layout_passes=False,
                                         use_tc_tiling_on_sc=False),
    scratch_types=[
        pltpu.VMEM((NCHUNK, CHUNK), jnp.int32),
        pltpu.VMEM((NCHUNK, CHUNK), jnp.int32),
        pltpu.VMEM((NCHUNK, CHUNK, EMBED), jnp.float32),
        pltpu.VMEM((NCHUNK, CHUNK, EMBED), jnp.float32),
        pltpu.VMEM((B_PER_W,), jnp.float32),
        pltpu.SemaphoreType.DMA,
    ],
)


def kernel(users, items, user_embed, item_embed):
    u_idx = users.astype(jnp.int32).reshape(NW, NCHUNK, CHUNK)
    i_idx = items.astype(jnp.int32).reshape(NW, NCHUNK, CHUNK)
    out = _sc_kernel(u_idx, i_idx, user_embed, item_embed)
    return out.reshape(BATCH)


# speedup vs baseline: 3.6211x; 3.6211x over previous
import jax
import jax.numpy as jnp
from jax import lax
from jax.experimental import pallas as pl
from jax.experimental.pallas import tpu as pltpu, tpu_sc as plsc

NUM_CORES = 2
NUM_SUBCORES = 16
NW = NUM_CORES * NUM_SUBCORES
BATCH = 16384
EMBED = 32
B_PER_W = BATCH // NW
LANE = 128
K = 8
NSTORE = B_PER_W // 16

_mesh = plsc.VectorSubcoreMesh(core_axis_name="c", subcore_axis_name="s",
                               num_cores=NUM_CORES, num_subcores=NUM_SUBCORES)

_DNUMS = lax.GatherDimensionNumbers(
    offset_dims=(), collapsed_slice_dims=(0,), start_index_map=(0,))


def _lane_sum(s, lanes):
    for sh in (8, 4, 2, 1):
        perm = lax.bitwise_xor(lanes, sh)[:, None]
        s = s + lax.gather(s, perm, _DNUMS, (1,),
                           mode=lax.GatherScatterMode.PROMISE_IN_BOUNDS)
    return s


def _sc_body(users_r, items_r, ue_r, ie_r, out_r,
             uidx_v, iidx_v, uslab_v, islab_v, out_v, sem):
    wid = lax.axis_index("s") * NUM_CORES + lax.axis_index("c")
    base = wid * B_PER_W

    pltpu.sync_copy(users_r.at[pl.ds(base, B_PER_W)], uidx_v)
    pltpu.sync_copy(items_r.at[pl.ds(base, B_PER_W)], iidx_v)

    lanes = lax.iota(jnp.int32, 16)

    def store_grp(g, _):
        o16 = pl.multiple_of(g * 16, 16)
        uvec = uidx_v[pl.ds(o16, 16)]
        ivec = iidx_v[pl.ds(o16, 16)]
        acc = jnp.zeros((16,), jnp.float32)
        for half in range(2):
            copies = []
            for kk in range(K):
                e = half * K + kk
                ub = pl.multiple_of((uvec[e] >> 7) * LANE, LANE)
                ib = pl.multiple_of((ivec[e] >> 7) * LANE, LANE)
                copies.append(pltpu.async_copy(
                    ue_r.at[:, pl.ds(ub, LANE)], uslab_v.at[kk], sem))
                copies.append(pltpu.async_copy(
                    ie_r.at[:, pl.ds(ib, LANE)], islab_v.at[kk], sem))
            for cp in copies:
                cp.wait()

            for kk in range(K):
                e = half * K + kk
                kv = jnp.full((16,), kk, jnp.int32)
                uc = jnp.full((16,), uvec[e] & (LANE - 1), jnp.int32)
                ic = jnp.full((16,), ivec[e] & (LANE - 1), jnp.int32)
                u0 = plsc.load_gather(uslab_v, [kv, lanes, uc])
                u1 = plsc.load_gather(uslab_v, [kv, lanes + 16, uc])
                v0 = plsc.load_gather(islab_v, [kv, lanes, ic])
                v1 = plsc.load_gather(islab_v, [kv, lanes + 16, ic])
                s = _lane_sum(u0 * v0 + u1 * v1, lanes)
                acc = jnp.where(lanes == e, s, acc)
        out_v[pl.ds(o16, 16)] = acc
        return 0

    lax.fori_loop(0, NSTORE, store_grp, 0)
    pltpu.sync_copy(out_v, out_r.at[pl.ds(base, B_PER_W)])


_sc_kernel = pl.kernel(
    _sc_body,
    out_type=jax.ShapeDtypeStruct((BATCH,), jnp.float32),
    mesh=_mesh,
    compiler_params=pltpu.CompilerParams(needs_layout_passes=False),
    scratch_types=[
        pltpu.VMEM((B_PER_W,), jnp.int32),
        pltpu.VMEM((B_PER_W,), jnp.int32),
        pltpu.VMEM((K, EMBED, LANE), jnp.float32),
        pltpu.VMEM((K, EMBED, LANE), jnp.float32),
        pltpu.VMEM((B_PER_W,), jnp.float32),
        pltpu.SemaphoreType.DMA,
    ],
)


def kernel(users, items, user_embed, item_embed):
    return _sc_kernel(users.astype(jnp.int32), items.astype(jnp.int32),
                      user_embed.T, item_embed.T)


# speedup vs baseline: 4.4147x; 1.2192x over previous
import jax
import jax.numpy as jnp
from jax import lax
from jax.experimental import pallas as pl
from jax.experimental.pallas import tpu as pltpu, tpu_sc as plsc

NUM_CORES = 2
NUM_SUBCORES = 16
NW = NUM_CORES * NUM_SUBCORES
BATCH = 16384
EMBED = 32
B_PER_W = BATCH // NW
LANE = 128
RING = 8
NITER = B_PER_W // RING

_mesh = plsc.VectorSubcoreMesh(core_axis_name="c", subcore_axis_name="s",
                               num_cores=NUM_CORES, num_subcores=NUM_SUBCORES)

_DNUMS = lax.GatherDimensionNumbers(
    offset_dims=(), collapsed_slice_dims=(0,), start_index_map=(0,))


def _lane_sum(s, lanes):
    for sh in (8, 4, 2, 1):
        perm = lax.bitwise_xor(lanes, sh)[:, None]
        s = s + lax.gather(s, perm, _DNUMS, (1,),
                           mode=lax.GatherScatterMode.PROMISE_IN_BOUNDS)
    return s


def _sc_body(users_r, items_r, ue_r, ie_r, out_r,
             uidx_v, iidx_v, uslab_v, islab_v, out_v, sem):
    wid = lax.axis_index("s") * NUM_CORES + lax.axis_index("c")
    base = wid * B_PER_W

    pltpu.sync_copy(users_r.at[pl.ds(base, B_PER_W)],
                    uidx_v.at[pl.ds(0, B_PER_W)])
    pltpu.sync_copy(items_r.at[pl.ds(base, B_PER_W)],
                    iidx_v.at[pl.ds(0, B_PER_W)])

    lanes = lax.iota(jnp.int32, 16)

    def fire(kk, uvi, ivi):
        ub = pl.multiple_of((uvi >> 7) * LANE, LANE)
        ib = pl.multiple_of((ivi >> 7) * LANE, LANE)
        pltpu.async_copy(ue_r.at[:, pl.ds(ub, LANE)], uslab_v.at[kk],
                         sem.at[kk])
        pltpu.async_copy(ie_r.at[:, pl.ds(ib, LANE)], islab_v.at[kk],
                         sem.at[kk])

    def drain(kk):
        pltpu.make_async_copy(
            ue_r.at[:, pl.ds(0, LANE)], uslab_v.at[kk], sem.at[kk]).wait()
        pltpu.make_async_copy(
            ie_r.at[:, pl.ds(0, LANE)], islab_v.at[kk], sem.at[kk]).wait()

    uvec0 = uidx_v[pl.ds(0, 16)]
    ivec0 = iidx_v[pl.ds(0, 16)]
    for kk in range(RING):
        fire(kk, uvec0[kk], ivec0[kk])

    def step(g, acc):
        o8 = pl.multiple_of(g * RING, 8)
        uvec = uidx_v[pl.ds(o8, 16)]
        ivec = iidx_v[pl.ds(o8, 16)]
        for kk in range(RING):
            drain(kk)
            uc = jnp.full((16,), uvec[kk] & (LANE - 1), jnp.int32)
            ic = jnp.full((16,), ivec[kk] & (LANE - 1), jnp.int32)
            kv = jnp.full((16,), kk, jnp.int32)
            u0 = plsc.load_gather(uslab_v, [kv, lanes, uc])
            u1 = plsc.load_gather(uslab_v, [kv, lanes + 16, uc])
            v0 = plsc.load_gather(islab_v, [kv, lanes, ic])
            v1 = plsc.load_gather(islab_v, [kv, lanes + 16, ic])

            @pl.when(g < NITER - 1)
            def _():
                fire(kk, uvec[RING + kk], ivec[RING + kk])

            s = _lane_sum(u0 * v0 + u1 * v1, lanes)
            e = (g % 2) * RING + kk
            acc = jnp.where(lanes == e, s, acc)

        @pl.when(g % 2 == 1)
        def _():
            out_v[pl.ds(pl.multiple_of((g // 2) * 16, 16), 16)] = acc
        return acc

    lax.fori_loop(0, NITER, step, jnp.zeros((16,), jnp.float32))
    pltpu.sync_copy(out_v, out_r.at[pl.ds(base, B_PER_W)])


_sc_kernel = pl.kernel(
    _sc_body,
    out_type=jax.ShapeDtypeStruct((BATCH,), jnp.float32),
    mesh=_mesh,
    compiler_params=pltpu.CompilerParams(needs_layout_passes=False),
    scratch_types=[
        pltpu.VMEM((B_PER_W + 16,), jnp.int32),
        pltpu.VMEM((B_PER_W + 16,), jnp.int32),
        pltpu.VMEM((RING, EMBED, LANE), jnp.float32),
        pltpu.VMEM((RING, EMBED, LANE), jnp.float32),
        pltpu.VMEM((B_PER_W,), jnp.float32),
        pltpu.SemaphoreType.DMA((RING,)),
    ],
)


def kernel(users, items, user_embed, item_embed):
    return _sc_kernel(users.astype(jnp.int32), items.astype(jnp.int32),
                      user_embed.T, item_embed.T)
